# baseline (speedup 1.0000x reference)
import functools

import jax
import jax.numpy as jnp
from jax import lax
from jax.experimental import pallas as pl
from jax.experimental.pallas import tpu as pltpu
from jax.experimental.pallas import tpu_sc as plsc

V_DIM = 100000
EMB_DIM = 64
BATCH = 4096
CTX = 20

NUM_CORES = 2
NUM_SUBCORES = 16
NUM_WORKERS = NUM_CORES * NUM_SUBCORES
FEATS_PER_W = EMB_DIM // NUM_WORKERS
LANES = 16
NB = 256
N_BLOCKS = BATCH // NB
INV_CTX = 1.0 / CTX


def _cbow_body(tbl_hbm, xt_hbm, out_hbm,
               tbl_v, xb0, xb1, acc_v, sem_t, sem_x0, sem_x1):
    wid = lax.axis_index("c") * NUM_SUBCORES + lax.axis_index("s")

    xbufs = (xb0, xb1)
    xsems = (sem_x0, sem_x1)

    for f in range(FEATS_PER_W):
        d = wid * FEATS_PER_W + f
        ct = pltpu.async_copy(tbl_hbm.at[d], tbl_v, sem_t)
        cx = [None] * N_BLOCKS
        cx[0] = pltpu.async_copy(
            xt_hbm.at[:, pl.ds(0, NB)], xbufs[0], xsems[0])
        ct.wait()
        for blk in range(N_BLOCKS):
            cx[blk].wait()
            if blk + 1 < N_BLOCKS:
                cx[blk + 1] = pltpu.async_copy(
                    xt_hbm.at[:, pl.ds((blk + 1) * NB, NB)],
                    xbufs[(blk + 1) % 2], xsems[(blk + 1) % 2])
            xb = xbufs[blk % 2]

            @pl.loop(0, NB // LANES)
            def _(c, blk=blk, xb=xb):
                s = plsc.load_gather(tbl_v, [xb[0, pl.ds(c * LANES, LANES)]])
                for p in range(1, CTX):
                    s = s + plsc.load_gather(
                        tbl_v, [xb[p, pl.ds(c * LANES, LANES)]])
                acc_v[pl.ds(blk * NB + c * LANES, LANES)] = s * INV_CTX

        pltpu.sync_copy(acc_v, out_hbm.at[d])


@jax.jit
def _cbow_sc(tbl_t, xt):
    mesh = plsc.VectorSubcoreMesh(core_axis_name="c", subcore_axis_name="s")
    kern = functools.partial(
        pl.kernel,
        out_type=jax.ShapeDtypeStruct((EMB_DIM, BATCH), jnp.float32),
        mesh=mesh,
        compiler_params=pltpu.CompilerParams(
            use_tc_tiling_on_sc=False, needs_layout_passes=False),
        scratch_types=[
            pltpu.VMEM((V_DIM,), jnp.float32),
            pltpu.VMEM((CTX, NB), jnp.int32),
            pltpu.VMEM((CTX, NB), jnp.int32),
            pltpu.VMEM((BATCH,), jnp.float32),
            pltpu.SemaphoreType.DMA,
            pltpu.SemaphoreType.DMA,
            pltpu.SemaphoreType.DMA,
        ],
    )(_cbow_body)
    return kern(tbl_t, xt)


def kernel(x, embeddings):
    out_t = _cbow_sc(embeddings.T, x.astype(jnp.int32).T)
    return out_t.T
